# Initial kernel scaffold; baseline (speedup 1.0000x reference)
import jax
import jax.numpy as jnp
from jax.experimental import pallas as pl


def kernel(coefficients, groups):
    raise NotImplementedError("write your pallas kernel here")


# speedup vs baseline: 161.0266x; 161.0266x over previous
import functools

import jax
import jax.numpy as jnp
from jax import lax
from jax.experimental import pallas as pl
from jax.experimental.pallas import tpu as pltpu
from jax.experimental.pallas import tpu_sc as plsc

N = 6_400_000
G = 100_000
G_PAD = 100_352
GSL = G_PAD // 16
NW = 32
EPW = N // NW
B1 = 20_000
B2 = 10_000
COEF = 0.1 * 0.01

_mesh = plsc.VectorSubcoreMesh(
    core_axis_name="c", subcore_axis_name="s", num_cores=2, num_subcores=16
)
_sc_params = pltpu.CompilerParams(needs_layout_passes=False)


@functools.partial(
    pl.kernel,
    out_type=jax.ShapeDtypeStruct((2, G_PAD), jnp.float32),
    mesh=_mesh,
    scratch_types=[
        pltpu.VMEM((B1,), jnp.float32),
        pltpu.VMEM((B1,), jnp.int32),
        pltpu.VMEM((GSL,), jnp.float32),
        pltpu.VMEM_SHARED((G_PAD,), jnp.float32),
    ],
    compiler_params=_sc_params,
)
def _sumsq_kernel(coef_hbm, groups_hbm, out_hbm, cbuf, gbuf, sbuf, acc_sh):
    c = lax.axis_index("c")
    s = lax.axis_index("s")
    wid = s * 2 + c

    zeros = jnp.zeros((16,), jnp.float32)

    @pl.loop(0, GSL // 16)
    def _(i):
        sbuf[pl.ds(i * 16, 16)] = zeros

    pltpu.sync_copy(sbuf, acc_sh.at[pl.ds(s * GSL, GSL)])
    plsc.subcore_barrier()

    for b in range(EPW // B1):
        base = wid * EPW + b * B1
        pltpu.sync_copy(coef_hbm.at[pl.ds(base, B1)], cbuf)
        pltpu.sync_copy(groups_hbm.at[pl.ds(base, B1)], gbuf)

        @pl.loop(0, B1 // 16)
        def _(i):
            x = cbuf[pl.ds(i * 16, 16)]
            cbuf[pl.ds(i * 16, 16)] = x * x

        pltpu.sync_copy(cbuf, acc_sh.at[gbuf], add=True)

    plsc.subcore_barrier()
    pltpu.sync_copy(acc_sh.at[pl.ds(s * GSL, GSL)], sbuf)
    pltpu.sync_copy(sbuf, out_hbm.at[c, pl.ds(s * GSL, GSL)])


def _factor_body(p_ref, f_ref):
    sumsq = p_ref[0:1, :] + p_ref[1:2, :]
    norm = jnp.sqrt(sumsq + 1e-12)
    f_ref[...] = jnp.maximum(1.0 - COEF / (norm + 1e-10), 0.0)


_factor_call = pl.pallas_call(
    _factor_body,
    out_shape=jax.ShapeDtypeStruct((1, G_PAD), jnp.float32),
)


@functools.partial(
    pl.kernel,
    out_type=jax.ShapeDtypeStruct((N,), jnp.float32),
    mesh=_mesh,
    scratch_types=[
        pltpu.VMEM((G_PAD,), jnp.float32),
        pltpu.VMEM((B2,), jnp.float32),
        pltpu.VMEM((B2,), jnp.int32),
    ],
    compiler_params=_sc_params,
)
def _apply_kernel(coef_hbm, groups_hbm, factor_hbm, out_hbm, fbuf, cbuf, gbuf):
    c = lax.axis_index("c")
    s = lax.axis_index("s")
    wid = s * 2 + c

    pltpu.sync_copy(factor_hbm, fbuf)

    for b in range(EPW // B2):
        base = wid * EPW + b * B2
        pltpu.sync_copy(coef_hbm.at[pl.ds(base, B2)], cbuf)
        pltpu.sync_copy(groups_hbm.at[pl.ds(base, B2)], gbuf)

        @pl.loop(0, B2 // 16)
        def _(i):
            g = gbuf[pl.ds(i * 16, 16)]
            f = plsc.load_gather(fbuf, [g])
            x = cbuf[pl.ds(i * 16, 16)]
            cbuf[pl.ds(i * 16, 16)] = x * f

        pltpu.sync_copy(cbuf, out_hbm.at[pl.ds(base, B2)])


def kernel(coefficients, groups):
    groups = groups.astype(jnp.int32)
    partials = _sumsq_kernel(coefficients, groups)
    factor = _factor_call(partials).reshape(G_PAD)
    return _apply_kernel(coefficients, groups, factor)
